# baseline (speedup 1.0000x reference)
import math

import jax
import jax.numpy as jnp
from jax.experimental import pallas as pl
from jax.experimental.pallas import tpu as pltpu

BS = 2
N_CTX = 2048
N_HEADS = 12
WIDTH = 64
N_TOK = 128
HPG = 4


def _split_hi_lo(x):
    hi = x.astype(jnp.bfloat16)
    lo = (x - hi.astype(jnp.float32)).astype(jnp.bfloat16)
    return hi, lo


def _one_head(oh, qp, kp, vp):
    scale = 1.0 / math.sqrt(N_TOK)
    qs = jnp.dot(qp.astype(jnp.bfloat16), oh, preferred_element_type=jnp.float32)
    ks = jnp.dot(kp.astype(jnp.bfloat16), oh, preferred_element_type=jnp.float32)
    kT = ks.T
    qsb = qs.astype(jnp.bfloat16)
    kTb = kT.astype(jnp.bfloat16)

    accs = [jnp.zeros((8, N_TOK), jnp.float32) for _ in range(N_TOK // 8)]
    for w in range(WIDTH):
        qrow = qsb[w:w + 1, :]
        col = kTb[:, w:w + 1]
        for ci in range(N_TOK // 8):
            d = jnp.abs(col[8 * ci:8 * ci + 8, :] - qrow)
            accs[ci] = accs[ci] + d.astype(jnp.float32)
    a = -scale * jnp.concatenate(accs, axis=0)
    a = a - jnp.max(a, axis=1, keepdims=True)
    e = jnp.exp(a)
    p = e / jnp.sum(e, axis=1, keepdims=True)

    vhi, vlo = _split_hi_lo(vp)
    vs = (jnp.dot(vhi, oh, preferred_element_type=jnp.float32)
          + jnp.dot(vlo, oh, preferred_element_type=jnp.float32))

    vshi, vslo = _split_hi_lo(vs)
    pb = p.astype(jnp.bfloat16)
    dn = (((1,), (1,)), ((), ()))
    return (jax.lax.dot_general(vshi, pb, dn, preferred_element_type=jnp.float32)
            + jax.lax.dot_general(vslo, pb, dn, preferred_element_type=jnp.float32))


def _attn_body(ir, qr, kr, vr, outr, oh_ref):
    b = pl.program_id(0)
    h = pl.program_id(1)

    @pl.when(jnp.logical_and(b == 0, h == 0))
    def _():
        ids = jax.lax.broadcasted_iota(jnp.int32, (N_CTX, N_TOK), 0)
        oh_ref[...] = (ids == ir[...]).astype(jnp.bfloat16)

    oh = oh_ref[...]
    for hh in range(HPG):
        outr[0, hh] = _one_head(oh, qr[0, hh], kr[0, hh], vr[0, hh])


def kernel(q, k, v, indx):
    qv = jnp.transpose(q, (0, 2, 3, 1))
    kv = jnp.transpose(k, (0, 2, 3, 1))
    vv = jnp.transpose(v, (0, 2, 3, 1))
    idx = indx.astype(jnp.int32).reshape(1, N_TOK)

    plane = pl.BlockSpec((1, HPG, WIDTH, N_CTX), lambda b, h: (b, h, 0, 0))
    y = pl.pallas_call(
        _attn_body,
        grid=(BS, N_HEADS // HPG),
        in_specs=[
            pl.BlockSpec((1, N_TOK), lambda b, h: (0, 0)),
            plane, plane, plane,
        ],
        out_specs=pl.BlockSpec((1, HPG, WIDTH, N_TOK), lambda b, h: (b, h, 0, 0)),
        out_shape=jax.ShapeDtypeStruct((BS, N_HEADS, WIDTH, N_TOK), jnp.float32),
        scratch_shapes=[pltpu.VMEM((N_CTX, N_TOK), jnp.bfloat16)],
    )(idx, qv, kv, vv)
    return jnp.transpose(y, (0, 3, 1, 2))


# speedup vs baseline: 1.1095x; 1.1095x over previous
import math

import jax
import jax.numpy as jnp
from jax.experimental import pallas as pl
from jax.experimental.pallas import tpu as pltpu

BS = 2
N_CTX = 2048
N_HEADS = 12
WIDTH = 64
N_TOK = 128
HPG = 2


def _split_hi_lo(x):
    hi = x.astype(jnp.bfloat16)
    lo = (x - hi.astype(jnp.float32)).astype(jnp.bfloat16)
    return hi, lo


def _select(oh, qp, kp, vp):
    qs = jnp.dot(qp.astype(jnp.bfloat16), oh, preferred_element_type=jnp.float32)
    ks = jnp.dot(kp.astype(jnp.bfloat16), oh, preferred_element_type=jnp.float32)
    kT = ks.T
    qsb = qs.astype(jnp.bfloat16)
    kTb = kT.astype(jnp.bfloat16)
    vhi, vlo = _split_hi_lo(vp)
    vs = (jnp.dot(vhi, oh, preferred_element_type=jnp.float32)
          + jnp.dot(vlo, oh, preferred_element_type=jnp.float32))
    return qsb, kTb, vs


def _softmax_l1(qsb, kTb):
    scale = 1.0 / math.sqrt(N_TOK)
    accs = [jnp.zeros((8, N_TOK), jnp.float32) for _ in range(N_TOK // 8)]
    for w in range(WIDTH):
        qrow = qsb[w:w + 1, :]
        col = kTb[:, w:w + 1]
        for ci in range(N_TOK // 8):
            d = jnp.abs(col[8 * ci:8 * ci + 8, :] - qrow)
            accs[ci] = accs[ci] + d.astype(jnp.float32)
    a = -scale * jnp.concatenate(accs, axis=0)
    a = a - jnp.max(a, axis=1, keepdims=True)
    e = jnp.exp(a)
    return e / jnp.sum(e, axis=1, keepdims=True)


def _combine(vs, p):
    vshi, vslo = _split_hi_lo(vs)
    pb = p.astype(jnp.bfloat16)
    dn = (((1,), (1,)), ((), ()))
    return (jax.lax.dot_general(vshi, pb, dn, preferred_element_type=jnp.float32)
            + jax.lax.dot_general(vslo, pb, dn, preferred_element_type=jnp.float32))


def _attn_body(ir, qr, kr, vr, outr, oh_ref):
    b = pl.program_id(0)
    h = pl.program_id(1)

    @pl.when(jnp.logical_and(b == 0, h == 0))
    def _():
        ids = jax.lax.broadcasted_iota(jnp.int32, (N_CTX, N_TOK), 0)
        oh_ref[...] = (ids == ir[...]).astype(jnp.bfloat16)

    oh = oh_ref[...]
    sel = [_select(oh, qr[0, hh], kr[0, hh], vr[0, hh]) for hh in range(HPG)]
    ps = [_softmax_l1(qsb, kTb) for qsb, kTb, _ in sel]
    for hh in range(HPG):
        outr[0, hh] = _combine(sel[hh][2], ps[hh])


def kernel(q, k, v, indx):
    qv = jnp.transpose(q, (0, 2, 3, 1))
    kv = jnp.transpose(k, (0, 2, 3, 1))
    vv = jnp.transpose(v, (0, 2, 3, 1))
    idx = indx.astype(jnp.int32).reshape(1, N_TOK)

    plane = pl.BlockSpec((1, HPG, WIDTH, N_CTX), lambda b, h: (b, h, 0, 0))
    y = pl.pallas_call(
        _attn_body,
        grid=(BS, N_HEADS // HPG),
        in_specs=[
            pl.BlockSpec((1, N_TOK), lambda b, h: (0, 0)),
            plane, plane, plane,
        ],
        out_specs=pl.BlockSpec((1, HPG, WIDTH, N_TOK), lambda b, h: (b, h, 0, 0)),
        out_shape=jax.ShapeDtypeStruct((BS, N_HEADS, WIDTH, N_TOK), jnp.float32),
        scratch_shapes=[pltpu.VMEM((N_CTX, N_TOK), jnp.bfloat16)],
    )(idx, qv, kv, vv)
    return jnp.transpose(y, (0, 3, 1, 2))


# speedup vs baseline: 1.1941x; 1.0763x over previous
import math

import jax
import jax.numpy as jnp
from jax.experimental import pallas as pl
from jax.experimental.pallas import tpu as pltpu

BS = 2
N_CTX = 2048
N_HEADS = 12
WIDTH = 64
N_TOK = 128
HPG = 6


def _split_hi_lo(x):
    hi = x.astype(jnp.bfloat16)
    lo = (x - hi.astype(jnp.float32)).astype(jnp.bfloat16)
    return hi, lo


def _select(oh, qp, kp, vp):
    qs = jnp.dot(qp.astype(jnp.bfloat16), oh, preferred_element_type=jnp.float32)
    ks = jnp.dot(kp.astype(jnp.bfloat16), oh, preferred_element_type=jnp.float32)
    kT = ks.T
    qsb = qs.astype(jnp.bfloat16)
    kTb = kT.astype(jnp.bfloat16)
    vhi, vlo = _split_hi_lo(vp)
    vs = (jnp.dot(vhi, oh, preferred_element_type=jnp.float32)
          + jnp.dot(vlo, oh, preferred_element_type=jnp.float32))
    return qsb, kTb, vs


def _softmax_l1(qsb, kTb):
    scale = 1.0 / math.sqrt(N_TOK)
    accs = [jnp.zeros((8, N_TOK), jnp.float32) for _ in range(N_TOK // 8)]
    for w in range(WIDTH):
        qrow = qsb[w:w + 1, :]
        col = kTb[:, w:w + 1]
        for ci in range(N_TOK // 8):
            d = jnp.abs(col[8 * ci:8 * ci + 8, :] - qrow)
            accs[ci] = accs[ci] + d.astype(jnp.float32)
    a = -scale * jnp.concatenate(accs, axis=0)
    a = a - jnp.max(a, axis=1, keepdims=True)
    e = jnp.exp(a)
    return e / jnp.sum(e, axis=1, keepdims=True)


def _combine(vs, p):
    vshi, vslo = _split_hi_lo(vs)
    pb = p.astype(jnp.bfloat16)
    dn = (((1,), (1,)), ((), ()))
    return (jax.lax.dot_general(vshi, pb, dn, preferred_element_type=jnp.float32)
            + jax.lax.dot_general(vslo, pb, dn, preferred_element_type=jnp.float32))


def _attn_body(ir, qr, kr, vr, outr, oh_ref):
    b = pl.program_id(0)
    h = pl.program_id(1)

    @pl.when(jnp.logical_and(b == 0, h == 0))
    def _():
        ids = jax.lax.broadcasted_iota(jnp.int32, (N_CTX, N_TOK), 0)
        oh_ref[...] = (ids == ir[...]).astype(jnp.bfloat16)

    oh = oh_ref[...]
    sel = [_select(oh, qr[0, hh], kr[0, hh], vr[0, hh]) for hh in range(HPG)]
    ps = [_softmax_l1(qsb, kTb) for qsb, kTb, _ in sel]
    for hh in range(HPG):
        outr[0, hh] = _combine(sel[hh][2], ps[hh])


def kernel(q, k, v, indx):
    qv = jnp.transpose(q, (0, 2, 3, 1))
    kv = jnp.transpose(k, (0, 2, 3, 1))
    vv = jnp.transpose(v, (0, 2, 3, 1))
    idx = indx.astype(jnp.int32).reshape(1, N_TOK)

    plane = pl.BlockSpec((1, HPG, WIDTH, N_CTX), lambda b, h: (b, h, 0, 0))
    y = pl.pallas_call(
        _attn_body,
        grid=(BS, N_HEADS // HPG),
        in_specs=[
            pl.BlockSpec((1, N_TOK), lambda b, h: (0, 0)),
            plane, plane, plane,
        ],
        out_specs=pl.BlockSpec((1, HPG, WIDTH, N_TOK), lambda b, h: (b, h, 0, 0)),
        out_shape=jax.ShapeDtypeStruct((BS, N_HEADS, WIDTH, N_TOK), jnp.float32),
        scratch_shapes=[pltpu.VMEM((N_CTX, N_TOK), jnp.bfloat16)],
    )(idx, qv, kv, vv)
    return jnp.transpose(y, (0, 3, 1, 2))


# speedup vs baseline: 1.1958x; 1.0014x over previous
import math

import jax
import jax.numpy as jnp
from jax.experimental import pallas as pl
from jax.experimental.pallas import tpu as pltpu

BS = 2
N_CTX = 2048
N_HEADS = 12
WIDTH = 64
N_TOK = 128
HPG = 4


def _split_hi_lo(x):
    hi = x.astype(jnp.bfloat16)
    lo = (x - hi.astype(jnp.float32)).astype(jnp.bfloat16)
    return hi, lo


def _select(oh, qp, kp, vp):
    qs = jnp.dot(qp.astype(jnp.bfloat16), oh, preferred_element_type=jnp.float32)
    ks = jnp.dot(kp.astype(jnp.bfloat16), oh, preferred_element_type=jnp.float32)
    kT = ks.T
    qsb = qs.astype(jnp.bfloat16)
    kTb = kT.astype(jnp.bfloat16)
    vhi, vlo = _split_hi_lo(vp)
    vs = (jnp.dot(vhi, oh, preferred_element_type=jnp.float32)
          + jnp.dot(vlo, oh, preferred_element_type=jnp.float32))
    return qsb, kTb, vs


def _softmax_l1(qsb, kTb):
    scale = 1.0 / math.sqrt(N_TOK)
    accs = [jnp.zeros((8, N_TOK), jnp.float32) for _ in range(N_TOK // 8)]
    for w in range(WIDTH):
        qrow = qsb[w:w + 1, :]
        col = kTb[:, w:w + 1]
        for ci in range(N_TOK // 8):
            d = jnp.abs(col[8 * ci:8 * ci + 8, :] - qrow)
            accs[ci] = accs[ci] + d.astype(jnp.float32)
    a = -scale * jnp.concatenate(accs, axis=0)
    a = a - jnp.max(a, axis=1, keepdims=True)
    e = jnp.exp(a)
    return e / jnp.sum(e, axis=1, keepdims=True)


def _combine(vs, p):
    vshi, vslo = _split_hi_lo(vs)
    pb = p.astype(jnp.bfloat16)
    dn = (((1,), (1,)), ((), ()))
    return (jax.lax.dot_general(vshi, pb, dn, preferred_element_type=jnp.float32)
            + jax.lax.dot_general(vslo, pb, dn, preferred_element_type=jnp.float32))


def _attn_body(ir, qr, kr, vr, outr, oh_ref):
    b = pl.program_id(0)
    h = pl.program_id(1)

    @pl.when(jnp.logical_and(b == 0, h == 0))
    def _():
        ids = jax.lax.broadcasted_iota(jnp.int32, (N_CTX, N_TOK), 0)
        oh_ref[...] = (ids == ir[...]).astype(jnp.bfloat16)

    oh = oh_ref[...]
    sel = [_select(oh, qr[0, hh], kr[0, hh], vr[0, hh]) for hh in range(HPG)]
    ps = [_softmax_l1(qsb, kTb) for qsb, kTb, _ in sel]
    for hh in range(HPG):
        outr[0, hh] = _combine(sel[hh][2], ps[hh])


def kernel(q, k, v, indx):
    qv = jnp.transpose(q, (0, 2, 3, 1))
    kv = jnp.transpose(k, (0, 2, 3, 1))
    vv = jnp.transpose(v, (0, 2, 3, 1))
    idx = indx.astype(jnp.int32).reshape(1, N_TOK)

    plane = pl.BlockSpec((1, HPG, WIDTH, N_CTX), lambda b, h: (b, h, 0, 0))
    y = pl.pallas_call(
        _attn_body,
        grid=(BS, N_HEADS // HPG),
        in_specs=[
            pl.BlockSpec((1, N_TOK), lambda b, h: (0, 0)),
            plane, plane, plane,
        ],
        out_specs=pl.BlockSpec((1, HPG, WIDTH, N_TOK), lambda b, h: (b, h, 0, 0)),
        out_shape=jax.ShapeDtypeStruct((BS, N_HEADS, WIDTH, N_TOK), jnp.float32),
        scratch_shapes=[pltpu.VMEM((N_CTX, N_TOK), jnp.bfloat16)],
    )(idx, qv, kv, vv)
    return jnp.transpose(y, (0, 3, 1, 2))


# speedup vs baseline: 1.2436x; 1.0400x over previous
import math

import jax
import jax.numpy as jnp
from jax.experimental import pallas as pl
from jax.experimental.pallas import tpu as pltpu

BS = 2
N_CTX = 2048
N_HEADS = 12
WIDTH = 64
N_TOK = 128
HPG = 4


def _split_hi_lo(x):
    hi = x.astype(jnp.bfloat16)
    lo = (x - hi.astype(jnp.float32)).astype(jnp.bfloat16)
    return hi, lo


def _select(oh, qp, kp, vp):
    qs = jnp.dot(qp.astype(jnp.bfloat16), oh, preferred_element_type=jnp.float32)
    ks = jnp.dot(kp.astype(jnp.bfloat16), oh, preferred_element_type=jnp.float32)
    kT = ks.T
    qsb = qs.astype(jnp.bfloat16)
    kTb = kT.astype(jnp.bfloat16)
    vhi, vlo = _split_hi_lo(vp)
    vs = (jnp.dot(vhi, oh, preferred_element_type=jnp.float32)
          + jnp.dot(vlo, oh, preferred_element_type=jnp.float32))
    return qsb, kTb, vs


def _softmax_l1_multi(heads):
    scale = 1.0 / math.sqrt(N_TOK)
    n = len(heads)
    accs = [[jnp.zeros((8, N_TOK), jnp.float32) for _ in range(N_TOK // 8)]
            for _ in range(n)]
    for w in range(WIDTH):
        for hh, (qsb, kTb) in enumerate(heads):
            qrow = qsb[w:w + 1, :]
            col = kTb[:, w:w + 1]
            for ci in range(N_TOK // 8):
                d = jnp.abs(col[8 * ci:8 * ci + 8, :] - qrow)
                accs[hh][ci] = accs[hh][ci] + d.astype(jnp.float32)
    ps = []
    for hh in range(n):
        a = -scale * jnp.concatenate(accs[hh], axis=0)
        a = a - jnp.max(a, axis=1, keepdims=True)
        e = jnp.exp(a)
        ps.append(e / jnp.sum(e, axis=1, keepdims=True))
    return ps


def _combine(vs, p):
    vshi, vslo = _split_hi_lo(vs)
    pb = p.astype(jnp.bfloat16)
    dn = (((1,), (1,)), ((), ()))
    return (jax.lax.dot_general(vshi, pb, dn, preferred_element_type=jnp.float32)
            + jax.lax.dot_general(vslo, pb, dn, preferred_element_type=jnp.float32))


def _attn_body(ir, qr, kr, vr, outr, oh_ref):
    b = pl.program_id(0)
    h = pl.program_id(1)

    @pl.when(jnp.logical_and(b == 0, h == 0))
    def _():
        ids = jax.lax.broadcasted_iota(jnp.int32, (N_CTX, N_TOK), 0)
        oh_ref[...] = (ids == ir[...]).astype(jnp.bfloat16)

    oh = oh_ref[...]
    sel = [_select(oh, qr[0, hh], kr[0, hh], vr[0, hh]) for hh in range(HPG)]
    ps = _softmax_l1_multi([(qsb, kTb) for qsb, kTb, _ in sel])
    for hh in range(HPG):
        outr[0, hh] = _combine(sel[hh][2], ps[hh])


def kernel(q, k, v, indx):
    qv = jnp.transpose(q, (0, 2, 3, 1))
    kv = jnp.transpose(k, (0, 2, 3, 1))
    vv = jnp.transpose(v, (0, 2, 3, 1))
    idx = indx.astype(jnp.int32).reshape(1, N_TOK)

    plane = pl.BlockSpec((1, HPG, WIDTH, N_CTX), lambda b, h: (b, h, 0, 0))
    y = pl.pallas_call(
        _attn_body,
        grid=(BS, N_HEADS // HPG),
        in_specs=[
            pl.BlockSpec((1, N_TOK), lambda b, h: (0, 0)),
            plane, plane, plane,
        ],
        out_specs=pl.BlockSpec((1, HPG, WIDTH, N_TOK), lambda b, h: (b, h, 0, 0)),
        out_shape=jax.ShapeDtypeStruct((BS, N_HEADS, WIDTH, N_TOK), jnp.float32),
        scratch_shapes=[pltpu.VMEM((N_CTX, N_TOK), jnp.bfloat16)],
    )(idx, qv, kv, vv)
    return jnp.transpose(y, (0, 3, 1, 2))
